# baseline (speedup 1.0000x reference)
import jax
import jax.numpy as jnp
from jax import lax
from jax.experimental import pallas as pl
from jax.experimental.pallas import tpu as pltpu
from jax.experimental.pallas import tpu_sc as plsc

_VOCAB = 100000
_D = 768
_B = 4
_S = 8192
_LANES = 16

_info = plsc.get_sparse_core_info()
_NC = _info.num_cores
_NS = _info.num_subcores
_NW = _NC * _NS
_S_PER_W = _S // _NW
_CHUNK = 8
_ROWS = _B * _CHUNK
_N_CHUNKS = _S_PER_W // _CHUNK
_QUADS = _N_CHUNKS // 4
_WROWS = 2 * _CHUNK
_ROW_VREGS = _D // _LANES
_HALF = _ROW_VREGS // 2


def _emb_body(ids_hbm, wte_hbm, wpe_hbm, out_hbm,
              idx_all, rb0, rb1, rb2, rb3, wb0, wb1,
              sg0, sg1, sg2, sg3, so0, so1, so2, so3, sw0, sw1):
    wid = lax.axis_index("s") * _NC + lax.axis_index("c")
    s0 = wid * _S_PER_W

    pltpu.sync_copy(
        ids_hbm.at[pl.ds(wid * _N_CHUNKS * _ROWS, _N_CHUNKS * _ROWS)],
        idx_all)

    def clamp(k):
        for t in range(2):
            sl = pl.ds(k * _ROWS + t * _LANES, _LANES)
            idx_all[sl] = jnp.minimum(idx_all[sl], _VOCAB - 1)

    rbufs = (rb0, rb1, rb2, rb3)
    wbufs = (wb0, wb1)
    sgs = (sg0, sg1, sg2, sg3)
    sos = (so0, so1, so2, so3)
    sws = (sw0, sw1)

    def g_start(k, x):
        pltpu.async_copy(
            wte_hbm.at[idx_all.at[pl.ds(k * _ROWS, _ROWS)]],
            rbufs[x], sgs[x])

    def g_wait(x):
        pltpu.make_async_copy(
            wte_hbm.at[pl.ds(0, _ROWS)], rbufs[x], sgs[x]).wait()

    def o_start(k, x):
        for b in range(_B):
            pltpu.async_copy(
                rbufs[x].at[pl.ds(b * _CHUNK, _CHUNK)],
                out_hbm.at[b, pl.ds(s0 + k * _CHUNK, _CHUNK)], sos[x])

    def o_wait(x):
        pltpu.make_async_copy(
            rbufs[x], out_hbm.at[0, pl.ds(0, _ROWS)], sos[x]).wait()

    def w_start(p, wp):
        pltpu.async_copy(
            wpe_hbm.at[pl.ds(s0 + p * _WROWS, _WROWS)], wbufs[wp], sws[wp])

    def w_wait(wp):
        pltpu.make_async_copy(
            wpe_hbm.at[pl.ds(0, _WROWS)], wbufs[wp], sws[wp]).wait()

    def vadd(x, wp, half):
        rb = rbufs[x]
        wb = wbufs[wp]

        def row_body(r, c2):
            wr = half * _CHUNK + r
            for h in range(2):
                w = [wb[wr, pl.ds((h * _HALF + c) * _LANES, _LANES)]
                     for c in range(_HALF)]
                for b in range(_B):
                    for c in range(_HALF):
                        sl = pl.ds((h * _HALF + c) * _LANES, _LANES)
                        row = b * _CHUNK + r
                        rb[row, sl] = rb[row, sl] + w[c]
            return c2
        lax.fori_loop(0, _CHUNK, row_body, 0)

    w_start(0, 0)
    w_start(1, 1)
    clamp(0)
    g_start(0, 0)
    clamp(1)
    g_start(1, 1)
    clamp(2)
    g_start(2, 2)

    def quad_body(j, carry):
        for q in range(4):
            k = 4 * j + q
            cur = q
            nx3 = (q + 3) % 4
            half = q % 2
            wp = q // 2
            if half == 0:
                w_wait(wp)
            g_wait(cur)
            vadd(cur, wp, half)
            if q == 0:
                @pl.when(j > 0)
                def _():
                    o_wait(nx3)
            else:
                o_wait(nx3)
            if q == 0:
                clamp(k + 3)
                g_start(k + 3, nx3)
            else:
                @pl.when(j < _QUADS - 1)
                def _():
                    clamp(k + 3)
                    g_start(k + 3, nx3)
            if half == 1:
                if q == 1:
                    @pl.when(j < _QUADS - 1)
                    def _():
                        w_start(2 * j + 2, 0)
                else:
                    @pl.when(j < _QUADS - 1)
                    def _():
                        w_start(2 * j + 3, 1)
            o_start(k, cur)
        return carry

    lax.fori_loop(0, _QUADS, quad_body, 0)
    o_wait(3)


def kernel(inputs, wte, wpe):
    mesh = plsc.VectorSubcoreMesh(core_axis_name="c", subcore_axis_name="s")
    f = pl.kernel(
        _emb_body,
        mesh=mesh,
        out_type=jax.ShapeDtypeStruct((_B, _S, _D), jnp.float32),
        scratch_types=[
            pltpu.VMEM((_N_CHUNKS * _ROWS,), jnp.int32),
            pltpu.VMEM((_ROWS, _D), jnp.float32),
            pltpu.VMEM((_ROWS, _D), jnp.float32),
            pltpu.VMEM((_ROWS, _D), jnp.float32),
            pltpu.VMEM((_ROWS, _D), jnp.float32),
            pltpu.VMEM((_WROWS, _D), jnp.float32),
            pltpu.VMEM((_WROWS, _D), jnp.float32),
            pltpu.SemaphoreType.DMA,
            pltpu.SemaphoreType.DMA,
            pltpu.SemaphoreType.DMA,
            pltpu.SemaphoreType.DMA,
            pltpu.SemaphoreType.DMA,
            pltpu.SemaphoreType.DMA,
            pltpu.SemaphoreType.DMA,
            pltpu.SemaphoreType.DMA,
            pltpu.SemaphoreType.DMA,
            pltpu.SemaphoreType.DMA,
        ],
    )
    ids_perm = inputs.reshape(_B, _S // _CHUNK, _CHUNK).transpose(
        1, 0, 2).reshape(-1)
    return f(ids_perm, wte, wpe)
